# baseline (speedup 1.0000x reference)
import jax
import jax.numpy as jnp
from jax import lax
from jax.experimental import pallas as pl
from jax.experimental.pallas import tpu as pltpu

_NE = 8192
_D = 32
_BLK = 256
_HALF = 4096
_N = 65536
_NB = _N // _BLK
_COMMIT = 0.25


def _vq_block(z_ref, zh_ref, rn_ref, wn_ref, w_ref, zq_ref, idx_ref, loss_ref):
    zb = z_ref[...]
    zbh = zh_ref[...]
    rn = rn_ref[...]
    w = w_ref[...]
    wn = wn_ref[...]

    def half_argmin(lo):
        wc = w[lo:lo + _HALF, :].astype(jnp.bfloat16)
        mm = lax.dot_general(zbh, wc, (((1,), (1,)), ((), ())),
                             preferred_element_type=jnp.float32)
        dist = (rn + wn[:, lo:lo + _HALF]) - 2.0 * mm
        cmin = jnp.min(dist, axis=1, keepdims=True)
        iot = lax.broadcasted_iota(jnp.int32, (_BLK, _HALF), 1)
        cidx = jnp.min(jnp.where(dist == cmin, iot, _NE),
                       axis=1, keepdims=True) + lo
        return cmin, cidx

    m0, i0 = half_argmin(0)
    m1, i1 = half_argmin(_HALF)
    r = m0.astype(jnp.bfloat16).astype(jnp.float32)
    take = m1 < r
    best_i = jnp.where(take, i1, i0)

    zq = jnp.zeros((_BLK, _D), jnp.float32)
    for lo in range(0, _NE, _HALF):
        wc = w[lo:lo + _HALF, :]
        iot = lax.broadcasted_iota(jnp.int32, (_BLK, _HALF), 1) + lo
        oh = (best_i == iot).astype(jnp.float32)
        zq = zq + lax.dot_general(oh, wc, (((1,), (0,)), ((), ())),
                                  precision=lax.Precision.HIGHEST,
                                  preferred_element_type=jnp.float32)
    idx_ref[...] = best_i
    zq_ref[...] = zb + (zq - zb)
    blk_sum = jnp.sum((zq - zb) ** 2, keepdims=True)

    @pl.when(pl.program_id(0) == 0)
    def _init():
        loss_ref[...] = jnp.zeros((1, 1), jnp.float32)

    loss_ref[...] += blk_sum

    @pl.when(pl.program_id(0) == _NB - 1)
    def _finish():
        loss_ref[...] = loss_ref[...] * ((1.0 + _COMMIT) / (_N * _D))


def kernel(z, W):
    b, c, h, w = z.shape
    z_channel_last = jnp.transpose(z, (0, 2, 3, 1))
    z_flat = z_channel_last.reshape(-1, _D)
    rn = jnp.sum(z_flat ** 2, axis=1, keepdims=True)
    wn = jnp.sum(W ** 2, axis=1, keepdims=True).T
    z_flat_bf16 = z_flat.astype(jnp.bfloat16)
    zq_flat, idx2, loss11 = pl.pallas_call(
        _vq_block,
        grid=(_NB,),
        in_specs=[
            pl.BlockSpec((_BLK, _D), lambda i: (i, 0)),
            pl.BlockSpec((_BLK, _D), lambda i: (i, 0)),
            pl.BlockSpec((_BLK, 1), lambda i: (i, 0)),
            pl.BlockSpec((1, _NE), lambda i: (0, 0)),
            pl.BlockSpec((_NE, _D), lambda i: (0, 0)),
        ],
        out_specs=[
            pl.BlockSpec((_BLK, _D), lambda i: (i, 0)),
            pl.BlockSpec((_BLK, 1), lambda i: (i, 0)),
            pl.BlockSpec((1, 1), lambda i: (0, 0)),
        ],
        out_shape=[
            jax.ShapeDtypeStruct((_N, _D), jnp.float32),
            jax.ShapeDtypeStruct((_N, 1), jnp.int32),
            jax.ShapeDtypeStruct((1, 1), jnp.float32),
        ],
        compiler_params=pltpu.CompilerParams(
            dimension_semantics=("arbitrary",),
        ),
    )(z_flat, z_flat_bf16, rn, wn, W)
    z_q_st = zq_flat.reshape(b, h, w, c).transpose(0, 3, 1, 2)
    return (z_q_st, loss11[0, 0], idx2.reshape(-1))


# speedup vs baseline: 2.1372x; 2.1372x over previous
import jax
import jax.numpy as jnp
from jax import lax
from jax.experimental import pallas as pl
from jax.experimental.pallas import tpu as pltpu

_NE = 8192
_D = 32
_BLK = 256
_HALF = 4096
_N = 65536
_NB = _N // _BLK
_COMMIT = 0.25


def _vq_block(z_ref, zh_ref, rn_ref, wn_ref, wh_ref, wlo_ref,
              zq_ref, idx_ref, loss_ref):
    zb = z_ref[...]
    zbh = zh_ref[...]
    rn = rn_ref[...]
    wh = wh_ref[...]
    wn = wn_ref[...]
    iot = lax.broadcasted_iota(jnp.int32, (_BLK, _HALF), 1)

    def half_argmin(lo):
        mm = lax.dot_general(zbh, wh[lo:lo + _HALF, :], (((1,), (1,)), ((), ())),
                             preferred_element_type=jnp.float32)
        dist = (rn + wn[:, lo:lo + _HALF]) - 2.0 * mm
        cmin = jnp.min(dist, axis=1, keepdims=True)
        cidx = jnp.min(jnp.where(dist == cmin, iot, _NE),
                       axis=1, keepdims=True) + lo
        return cmin, cidx

    m0, i0 = half_argmin(0)
    m1, i1 = half_argmin(_HALF)
    r = m0.astype(jnp.bfloat16).astype(jnp.float32)
    take = m1 < r
    best_i = jnp.where(take, i1, i0)

    zq = jnp.zeros((_BLK, _D), jnp.float32)
    for lo in range(0, _NE, _HALF):
        oh = ((best_i - lo) == iot).astype(jnp.bfloat16)
        hi = lax.dot_general(oh, wh[lo:lo + _HALF, :], (((1,), (0,)), ((), ())),
                             preferred_element_type=jnp.float32)
        lo_p = lax.dot_general(oh, wlo_ref[lo:lo + _HALF, :], (((1,), (0,)), ((), ())),
                               preferred_element_type=jnp.float32)
        zq = zq + (hi + lo_p)
    idx_ref[...] = best_i
    zq_ref[...] = zb + (zq - zb)
    blk_sum = jnp.sum((zq - zb) ** 2, keepdims=True)

    @pl.when(pl.program_id(0) == 0)
    def _init():
        loss_ref[...] = jnp.zeros((1, 1), jnp.float32)

    loss_ref[...] += blk_sum

    @pl.when(pl.program_id(0) == _NB - 1)
    def _finish():
        loss_ref[...] = loss_ref[...] * ((1.0 + _COMMIT) / (_N * _D))


def kernel(z, W):
    b, c, h, w = z.shape
    z_channel_last = jnp.transpose(z, (0, 2, 3, 1))
    z_flat = z_channel_last.reshape(-1, _D)
    rn = jnp.sum(z_flat ** 2, axis=1, keepdims=True)
    wn = jnp.sum(W ** 2, axis=1, keepdims=True).T
    z_flat_bf16 = z_flat.astype(jnp.bfloat16)
    w_hi = W.astype(jnp.bfloat16)
    w_lo = (W - w_hi.astype(jnp.float32)).astype(jnp.bfloat16)
    zq_flat, idx2, loss11 = pl.pallas_call(
        _vq_block,
        grid=(_NB,),
        in_specs=[
            pl.BlockSpec((_BLK, _D), lambda i: (i, 0)),
            pl.BlockSpec((_BLK, _D), lambda i: (i, 0)),
            pl.BlockSpec((_BLK, 1), lambda i: (i, 0)),
            pl.BlockSpec((1, _NE), lambda i: (0, 0)),
            pl.BlockSpec((_NE, _D), lambda i: (0, 0)),
            pl.BlockSpec((_NE, _D), lambda i: (0, 0)),
        ],
        out_specs=[
            pl.BlockSpec((_BLK, _D), lambda i: (i, 0)),
            pl.BlockSpec((_BLK, 1), lambda i: (i, 0)),
            pl.BlockSpec((1, 1), lambda i: (0, 0)),
        ],
        out_shape=[
            jax.ShapeDtypeStruct((_N, _D), jnp.float32),
            jax.ShapeDtypeStruct((_N, 1), jnp.int32),
            jax.ShapeDtypeStruct((1, 1), jnp.float32),
        ],
        compiler_params=pltpu.CompilerParams(
            dimension_semantics=("arbitrary",),
        ),
    )(z_flat, z_flat_bf16, rn, wn, w_hi, w_lo)
    z_q_st = zq_flat.reshape(b, h, w, c).transpose(0, 3, 1, 2)
    return (z_q_st, loss11[0, 0], idx2.reshape(-1))


# speedup vs baseline: 3.3692x; 1.5764x over previous
import functools

import jax
import jax.numpy as jnp
from jax import lax
from jax.experimental import pallas as pl
from jax.experimental.pallas import tpu as pltpu
from jax.experimental.pallas import tpu_sc as plsc

_NE = 8192
_D = 32
_BLK = 256
_HALF = 4096
_N = 65536
_NB = _N // _BLK
_COMMIT = 0.25

_NW = 32
_RPW = _N // _NW
_GCH = 128

_BLK_B = 2048
_NB_B = _N // _BLK_B


def _argmin_block(zh_ref, rn_ref, wn_ref, wh_ref, idx_ref):
    zbh = zh_ref[...]
    rn = rn_ref[...]
    wh = wh_ref[...]
    wn = wn_ref[...]
    iot = lax.broadcasted_iota(jnp.int32, (_BLK, _HALF), 1)

    def half_argmin(lo):
        mm = lax.dot_general(zbh, wh[lo:lo + _HALF, :], (((1,), (1,)), ((), ())),
                             preferred_element_type=jnp.float32)
        dist = (rn + wn[:, lo:lo + _HALF]) - 2.0 * mm
        cmin = jnp.min(dist, axis=1, keepdims=True)
        cidx = jnp.min(jnp.where(dist == cmin, iot, _NE),
                       axis=1, keepdims=True) + lo
        return cmin, cidx

    m0, i0 = half_argmin(0)
    m1, i1 = half_argmin(_HALF)
    r = m0.astype(jnp.bfloat16).astype(jnp.float32)
    idx_ref[...] = jnp.where(m1 < r, i1, i0)


def _sc_gather(idx_hbm, w_hbm, out_hbm, idx_v, rows_v, sem):
    wid = lax.axis_index("s") * 2 + lax.axis_index("c")
    rbase = wid * (_RPW // _GCH)
    pltpu.sync_copy(idx_hbm.at[pl.ds(rbase, _RPW // _GCH)], idx_v)
    copies = []
    for j in range(_RPW // _GCH):
        copies.append(pltpu.async_copy(
            w_hbm.at[idx_v.at[j]], rows_v.at[pl.ds(j * _GCH, _GCH)], sem))
    for cp in copies:
        cp.wait()
    pltpu.sync_copy(rows_v, out_hbm.at[pl.ds(wid * _RPW, _RPW)])


def _st_loss_block(z_ref, zq_ref, out_ref, loss_ref):
    zb = z_ref[...]
    zq = zq_ref[...]
    out_ref[...] = zb + (zq - zb)
    blk_sum = jnp.sum((zq - zb) ** 2, keepdims=True)

    @pl.when(pl.program_id(0) == 0)
    def _init():
        loss_ref[...] = jnp.zeros((1, 1), jnp.float32)

    loss_ref[...] += blk_sum

    @pl.when(pl.program_id(0) == _NB_B - 1)
    def _finish():
        loss_ref[...] = loss_ref[...] * ((1.0 + _COMMIT) / (_N * _D))


def kernel(z, W):
    b, c, h, w = z.shape
    z_channel_last = jnp.transpose(z, (0, 2, 3, 1))
    z_flat = z_channel_last.reshape(-1, _D)
    rn = jnp.sum(z_flat ** 2, axis=1, keepdims=True)
    wn = jnp.sum(W ** 2, axis=1, keepdims=True).T
    z_flat_bf16 = z_flat.astype(jnp.bfloat16)
    w_hi = W.astype(jnp.bfloat16)

    idx2 = pl.pallas_call(
        _argmin_block,
        grid=(_NB,),
        in_specs=[
            pl.BlockSpec((_BLK, _D), lambda i: (i, 0)),
            pl.BlockSpec((_BLK, 1), lambda i: (i, 0)),
            pl.BlockSpec((1, _NE), lambda i: (0, 0)),
            pl.BlockSpec((_NE, _D), lambda i: (0, 0)),
        ],
        out_specs=pl.BlockSpec((_BLK, 1), lambda i: (i, 0)),
        out_shape=jax.ShapeDtypeStruct((_N, 1), jnp.int32),
        compiler_params=pltpu.CompilerParams(
            dimension_semantics=("arbitrary",),
        ),
    )(z_flat_bf16, rn, wn, w_hi)

    idx2d = idx2.reshape(_N // _GCH, _GCH)

    sc_gather = functools.partial(
        pl.kernel,
        mesh=plsc.VectorSubcoreMesh(core_axis_name="c", subcore_axis_name="s"),
        out_type=jax.ShapeDtypeStruct((_N, _D), jnp.float32),
        scratch_types=[
            pltpu.VMEM((_RPW // _GCH, _GCH), jnp.int32),
            pltpu.VMEM((_RPW, _D), jnp.float32),
            pltpu.SemaphoreType.DMA,
        ],
        compiler_params=pltpu.CompilerParams(use_tc_tiling_on_sc=False),
    )(_sc_gather)
    zq_flat = sc_gather(idx2d, W)

    zq_st_flat, loss11 = pl.pallas_call(
        _st_loss_block,
        grid=(_NB_B,),
        in_specs=[
            pl.BlockSpec((_BLK_B, _D), lambda i: (i, 0)),
            pl.BlockSpec((_BLK_B, _D), lambda i: (i, 0)),
        ],
        out_specs=[
            pl.BlockSpec((_BLK_B, _D), lambda i: (i, 0)),
            pl.BlockSpec((1, 1), lambda i: (0, 0)),
        ],
        out_shape=[
            jax.ShapeDtypeStruct((_N, _D), jnp.float32),
            jax.ShapeDtypeStruct((1, 1), jnp.float32),
        ],
        compiler_params=pltpu.CompilerParams(
            dimension_semantics=("arbitrary",),
        ),
    )(z_flat, zq_flat)

    z_q_st = zq_st_flat.reshape(b, h, w, c).transpose(0, 3, 1, 2)
    return (z_q_st, loss11[0, 0], idx2.reshape(-1))


# speedup vs baseline: 3.4087x; 1.0117x over previous
import functools

import jax
import jax.numpy as jnp
from jax import lax
from jax.experimental import pallas as pl
from jax.experimental.pallas import tpu as pltpu
from jax.experimental.pallas import tpu_sc as plsc

_NE = 8192
_D = 32
_BLK = 256
_HALF = 4096
_N = 65536
_NB = _N // _BLK
_COMMIT = 0.25

_NW = 32
_RPW = _N // _NW
_GCH = 128
_CCH = 512
_DP = 128

_BLK_B = 2048
_NB_B = _N // _BLK_B


def _argmin_block(zh_ref, rn_ref, wn_ref, wh_ref, idx_ref):
    zbh = zh_ref[...]
    rn = rn_ref[...]
    wh = wh_ref[...]
    wn = wn_ref[...]
    iot = lax.broadcasted_iota(jnp.int32, (_BLK, _HALF), 1)

    def half_argmin(lo):
        mm = lax.dot_general(zbh, wh[lo:lo + _HALF, :], (((1,), (1,)), ((), ())),
                             preferred_element_type=jnp.float32)
        dist = (rn + wn[:, lo:lo + _HALF]) - 2.0 * mm
        cmin = jnp.min(dist, axis=1, keepdims=True)
        cidx = jnp.min(jnp.where(dist == cmin, iot, _NE),
                       axis=1, keepdims=True) + lo
        return cmin, cidx

    m0, i0 = half_argmin(0)
    m1, i1 = half_argmin(_HALF)
    r = m0.astype(jnp.bfloat16).astype(jnp.float32)
    best = jnp.where(m1 < r, i1, i0)
    idx_ref[...] = best.reshape(1, _BLK // _GCH, _GCH)


def _sc_gather(idx_hbm, w_hbm, out_hbm, idx_v, rows_v, sem):
    wid = lax.axis_index("s") * 2 + lax.axis_index("c")
    rbase = wid * (_RPW // _GCH)
    pltpu.sync_copy(idx_hbm.at[pl.ds(rbase, _RPW // _GCH)], idx_v)
    for ch in range(_RPW // _CCH):
        copies = []
        for j in range(_CCH // _GCH):
            g = ch * (_CCH // _GCH) + j
            copies.append(pltpu.async_copy(
                w_hbm.at[idx_v.at[g]], rows_v.at[pl.ds(j * _GCH, _GCH)], sem))
        for cp in copies:
            cp.wait()
        pltpu.sync_copy(rows_v,
                        out_hbm.at[pl.ds(wid * _RPW + ch * _CCH, _CCH)])


def _st_loss_block(z_ref, zq_ref, out_ref, loss_ref):
    zb = z_ref[...]
    zq = zq_ref[:, :_D]
    out_ref[...] = zb + (zq - zb)
    blk_sum = jnp.sum((zq - zb) ** 2, keepdims=True)

    @pl.when(pl.program_id(0) == 0)
    def _init():
        loss_ref[...] = jnp.zeros((1, 1), jnp.float32)

    loss_ref[...] += blk_sum

    @pl.when(pl.program_id(0) == _NB_B - 1)
    def _finish():
        loss_ref[...] = loss_ref[...] * ((1.0 + _COMMIT) / (_N * _D))


def kernel(z, W):
    b, c, h, w = z.shape
    z_channel_last = jnp.transpose(z, (0, 2, 3, 1))
    z_flat = z_channel_last.reshape(-1, _D)
    rn = jnp.sum(z_flat ** 2, axis=1, keepdims=True)
    wn = jnp.sum(W ** 2, axis=1, keepdims=True).T
    z_flat_bf16 = z_flat.astype(jnp.bfloat16)
    w_hi = W.astype(jnp.bfloat16)

    idx2 = pl.pallas_call(
        _argmin_block,
        grid=(_NB,),
        in_specs=[
            pl.BlockSpec((_BLK, _D), lambda i: (i, 0)),
            pl.BlockSpec((_BLK, 1), lambda i: (i, 0)),
            pl.BlockSpec((1, _NE), lambda i: (0, 0)),
            pl.BlockSpec((_NE, _D), lambda i: (0, 0)),
        ],
        out_specs=pl.BlockSpec((1, _BLK // _GCH, _GCH), lambda i: (i, 0, 0)),
        out_shape=jax.ShapeDtypeStruct((_NB, _BLK // _GCH, _GCH), jnp.int32),
        compiler_params=pltpu.CompilerParams(
            dimension_semantics=("arbitrary",),
        ),
    )(z_flat_bf16, rn, wn, w_hi)
    idx2 = idx2.reshape(_N // _GCH, _GCH)

    w_pad = jnp.pad(W, ((0, 0), (0, _DP - _D)))

    sc_gather = functools.partial(
        pl.kernel,
        mesh=plsc.VectorSubcoreMesh(core_axis_name="c", subcore_axis_name="s"),
        out_type=jax.ShapeDtypeStruct((_N, _DP), jnp.float32),
        scratch_types=[
            pltpu.VMEM((_RPW // _GCH, _GCH), jnp.int32),
            pltpu.VMEM((_CCH, _DP), jnp.float32),
            pltpu.SemaphoreType.DMA,
        ],
    )(_sc_gather)
    zq_flat = sc_gather(idx2, w_pad)

    zq_st_flat, loss11 = pl.pallas_call(
        _st_loss_block,
        grid=(_NB_B,),
        in_specs=[
            pl.BlockSpec((_BLK_B, _D), lambda i: (i, 0)),
            pl.BlockSpec((_BLK_B, _DP), lambda i: (i, 0)),
        ],
        out_specs=[
            pl.BlockSpec((_BLK_B, _D), lambda i: (i, 0)),
            pl.BlockSpec((1, 1), lambda i: (0, 0)),
        ],
        out_shape=[
            jax.ShapeDtypeStruct((_N, _D), jnp.float32),
            jax.ShapeDtypeStruct((1, 1), jnp.float32),
        ],
        compiler_params=pltpu.CompilerParams(
            dimension_semantics=("arbitrary",),
        ),
    )(z_flat, zq_flat)

    z_q_st = zq_st_flat.reshape(b, h, w, c).transpose(0, 3, 1, 2)
    return (z_q_st, loss11[0, 0], idx2.reshape(-1))


# speedup vs baseline: 3.4559x; 1.0138x over previous
import functools

import jax
import jax.numpy as jnp
from jax import lax
from jax.experimental import pallas as pl
from jax.experimental.pallas import tpu as pltpu
from jax.experimental.pallas import tpu_sc as plsc

_NE = 8192
_D = 32
_BLK = 256
_HALF = 4096
_N = 65536
_NB = _N // _BLK
_COMMIT = 0.25

_NW = 32
_RPW = _N // _NW
_GCH = 128
_CCH = 512
_DP = 128

_HW = 1024
_NB_B = 64


def _argmin_block(zh_ref, rn_ref, wn_ref, wh_ref, idx_ref):
    zbh = zh_ref[...]
    rn = rn_ref[...]
    wh = wh_ref[...]
    wn = wn_ref[...]
    iot = lax.broadcasted_iota(jnp.int32, (_BLK, _HALF), 1)

    def half_argmin(lo):
        mm = lax.dot_general(zbh, wh[lo:lo + _HALF, :], (((1,), (1,)), ((), ())),
                             preferred_element_type=jnp.float32)
        dist = (rn + wn[:, lo:lo + _HALF]) - 2.0 * mm
        cmin = jnp.min(dist, axis=1, keepdims=True)
        cidx = jnp.min(jnp.where(dist == cmin, iot, _NE),
                       axis=1, keepdims=True) + lo
        return cmin, cidx

    m0, i0 = half_argmin(0)
    m1, i1 = half_argmin(_HALF)
    r = m0.astype(jnp.bfloat16).astype(jnp.float32)
    best = jnp.where(m1 < r, i1, i0)
    idx_ref[...] = best.reshape(1, _BLK // _GCH, _GCH)


def _sc_gather(idx_hbm, w_hbm, out_hbm, idx_v, rows_v, sem):
    wid = lax.axis_index("s") * 2 + lax.axis_index("c")
    rbase = wid * (_RPW // _GCH)
    pltpu.sync_copy(idx_hbm.at[pl.ds(rbase, _RPW // _GCH)], idx_v)
    for ch in range(_RPW // _CCH):
        copies = []
        for j in range(_CCH // _GCH):
            g = ch * (_CCH // _GCH) + j
            copies.append(pltpu.async_copy(
                w_hbm.at[idx_v.at[g]], rows_v.at[pl.ds(j * _GCH, _GCH)], sem))
        for cp in copies:
            cp.wait()
        pltpu.sync_copy(rows_v,
                        out_hbm.at[pl.ds(wid * _RPW + ch * _CCH, _CCH)])


def _st_loss_block(z_ref, zq_ref, out_ref, loss_ref):
    zb = z_ref[0]
    zq = jnp.transpose(zq_ref[:, :_D])
    out_ref[0] = zb + (zq - zb)
    blk_sum = jnp.sum((zq - zb) ** 2, keepdims=True)

    @pl.when(pl.program_id(0) == 0)
    def _init():
        loss_ref[...] = jnp.zeros((1, 1), jnp.float32)

    loss_ref[...] += blk_sum

    @pl.when(pl.program_id(0) == _NB_B - 1)
    def _finish():
        loss_ref[...] = loss_ref[...] * ((1.0 + _COMMIT) / (_N * _D))


def kernel(z, W):
    b, c, h, w = z.shape
    z_channel_last = jnp.transpose(z, (0, 2, 3, 1))
    z_flat = z_channel_last.reshape(-1, _D)
    rn = jnp.sum(z_flat ** 2, axis=1, keepdims=True)
    wn = jnp.sum(W ** 2, axis=1, keepdims=True).T
    z_flat_bf16 = z_flat.astype(jnp.bfloat16)
    w_hi = W.astype(jnp.bfloat16)

    idx2 = pl.pallas_call(
        _argmin_block,
        grid=(_NB,),
        in_specs=[
            pl.BlockSpec((_BLK, _D), lambda i: (i, 0)),
            pl.BlockSpec((_BLK, 1), lambda i: (i, 0)),
            pl.BlockSpec((1, _NE), lambda i: (0, 0)),
            pl.BlockSpec((_NE, _D), lambda i: (0, 0)),
        ],
        out_specs=pl.BlockSpec((1, _BLK // _GCH, _GCH), lambda i: (i, 0, 0)),
        out_shape=jax.ShapeDtypeStruct((_NB, _BLK // _GCH, _GCH), jnp.int32),
        compiler_params=pltpu.CompilerParams(
            dimension_semantics=("arbitrary",),
        ),
    )(z_flat_bf16, rn, wn, w_hi)
    idx2 = idx2.reshape(_N // _GCH, _GCH)

    w_pad = jnp.pad(W, ((0, 0), (0, _DP - _D)))

    sc_gather = functools.partial(
        pl.kernel,
        mesh=plsc.VectorSubcoreMesh(core_axis_name="c", subcore_axis_name="s"),
        out_type=jax.ShapeDtypeStruct((_N, _DP), jnp.float32),
        scratch_types=[
            pltpu.VMEM((_RPW // _GCH, _GCH), jnp.int32),
            pltpu.VMEM((_CCH, _DP), jnp.float32),
            pltpu.SemaphoreType.DMA,
        ],
    )(_sc_gather)
    zq_flat = sc_gather(idx2, w_pad)

    z_native = z.reshape(b, _D, _HW)
    zq_st_n, loss11 = pl.pallas_call(
        _st_loss_block,
        grid=(_NB_B,),
        in_specs=[
            pl.BlockSpec((1, _D, _HW), lambda i: (i, 0, 0)),
            pl.BlockSpec((_HW, _DP), lambda i: (i, 0)),
        ],
        out_specs=[
            pl.BlockSpec((1, _D, _HW), lambda i: (i, 0, 0)),
            pl.BlockSpec((1, 1), lambda i: (0, 0)),
        ],
        out_shape=[
            jax.ShapeDtypeStruct((b, _D, _HW), jnp.float32),
            jax.ShapeDtypeStruct((1, 1), jnp.float32),
        ],
        compiler_params=pltpu.CompilerParams(
            dimension_semantics=("arbitrary",),
        ),
    )(z_native, zq_flat)

    z_q_st = zq_st_n.reshape(b, c, h, w)
    return (z_q_st, loss11[0, 0], idx2.reshape(-1))
